# baseline (speedup 1.0000x reference)
import jax
import jax.numpy as jnp
from jax import lax
from jax.experimental import pallas as pl
from jax.experimental.pallas import tpu as pltpu
from jax.experimental.pallas import tpu_sc as plsc

N = 10000
C = 128
K = 32
NP = 10240
AUGW = 144
NW = 32
RPW = 313
EPT = 625
NNR = NW * RPW
INF = 3.0e38
PINF = 0x7F61B1E6
MASK8 = -256


def _dist_body(xr_ref, xt_ref, out_ref, aux_ref):
    a = xr_ref[...]
    sa = jnp.sum(a * a, axis=1, keepdims=True)
    mm = None
    for l in range(16):
        b = xt_ref[:, pl.ds(l * 640, 640)]
        sb = jnp.sum(b * b, axis=0, keepdims=True)
        g = jnp.dot(a, b, preferred_element_type=jnp.float32)
        d = sa + sb - 2.0 * g
        if l == 15:
            col = lax.broadcasted_iota(jnp.int32, d.shape, 1)
            d = jnp.where(col + 9600 < N, d, INF)
        out_ref[:, l, :] = d
        dp = (lax.bitcast_convert_type(d, jnp.int32) & MASK8) | l
        mm = dp if mm is None else jnp.minimum(mm, dp)
    rb = a.shape[0]
    pinf128 = jnp.full((rb, 128), PINF, jnp.int32)
    for t in range(5):
        aux_ref[:, t, :] = mm[:, t * 128:(t + 1) * 128]
    aux_ref[:, 5, :] = pinf128
    aux_ref[:, 7, :] = pinf128
    mm768 = jnp.concatenate([mm, pinf128], axis=1)
    mb = None
    for l in range(16):
        w = mm768[:, l * 48:(l + 1) * 48] | (l << 4)
        mb = w if mb is None else jnp.minimum(mb, w)
    aux_ref[:, 6, :] = jnp.concatenate(
        [mb, jnp.full((rb, 80), PINF, jnp.int32)], axis=1)


def _build_dist(xpad, xpadT):
    RB = 128
    return pl.pallas_call(
        _dist_body,
        grid=(NP // RB,),
        in_specs=[pl.BlockSpec((RB, C), lambda i: (i, 0)),
                  pl.BlockSpec((C, NP), lambda i: (0, 0))],
        out_specs=[pl.BlockSpec((RB, 16, 640), lambda i: (i, 0, 0)),
                   pl.BlockSpec((RB, 8, 128), lambda i: (i, 0, 0))],
        out_shape=[jax.ShapeDtypeStruct((NP, 16, 640), jnp.float32),
                   jax.ShapeDtypeStruct((NP, 8, 128), jnp.int32)],
    )(xpad, xpadT)


def _select_body(d_hbm, aux_hbm, nn_hbm, rowbuf, auxbuf, nnbuf,
                 semr0, sema0, semr1, sema1):
    cid = lax.axis_index("c")
    sid = lax.axis_index("s")
    wid = sid * 2 + cid
    base = wid * RPW
    cnt = jnp.minimum(RPW, N - base)
    hi = base + cnt
    iota = lax.iota(jnp.int32, 16)
    big = jnp.int32(9999)
    inf16 = jnp.full((16,), INF, jnp.float32)
    pinf16 = jnp.full((16,), PINF, jnp.int32)
    lane0 = iota == 0

    def startp(r0, bp, sr, sa2):
        pltpu.async_copy(d_hbm.at[r0], rowbuf.at[bp], sr)
        pltpu.async_copy(aux_hbm.at[r0], auxbuf.at[bp], sa2)

        @pl.when(r0 + 1 < hi)
        def _():
            pltpu.async_copy(d_hbm.at[r0 + 1], rowbuf.at[bp + 1], sr)
            pltpu.async_copy(aux_hbm.at[r0 + 1], auxbuf.at[bp + 1], sa2)

    def waitp(r0, bp, sr, sa2):
        pltpu.make_async_copy(d_hbm.at[base], rowbuf.at[bp], sr).wait()
        pltpu.make_async_copy(aux_hbm.at[base], auxbuf.at[bp], sa2).wait()

        @pl.when(r0 + 1 < hi)
        def _():
            pltpu.make_async_copy(d_hbm.at[base], rowbuf.at[bp + 1],
                                  sr).wait()
            pltpu.make_async_copy(aux_hbm.at[base], auxbuf.at[bp + 1],
                                  sa2).wait()

    def one_ext(row, aux, it, nn_lo, nn_hi):
        v0 = aux[6, pl.ds(0, 16)]
        v1 = aux[6, pl.ds(16, 16)]
        v2 = aux[6, pl.ds(32, 16)]
        gmin = jnp.min(jnp.minimum(jnp.minimum(v0, v1), v2))
        cand = jnp.where(v0 == gmin, iota, big)
        cand = jnp.minimum(cand, jnp.where(v1 == gmin, iota + 16, big))
        cand = jnp.minimum(cand, jnp.where(v2 == gmin, iota + 32, big))
        mstar = jnp.min(cand)
        l2 = lax.shift_right_logical(gmin, 4) & 0xF
        g = mstar + 48 * l2
        cstar = gmin & 0xF
        ghi = jnp.full((16,), lax.shift_right_logical(g, 7), jnp.int32)
        glo = jnp.full((16,), g & 127, jnp.int32)
        u = plsc.load_gather(row, [iota, jnp.full((16,), g, jnp.int32)])
        colidx = cstar * 640 + g
        nn_lo = jnp.where(iota == it, colidx, nn_lo)
        nn_hi = jnp.where(iota == it - 16, colidx, nn_hi)
        plsc.store_scatter(row, [jnp.full((16,), cstar, jnp.int32),
                                 jnp.full((16,), g, jnp.int32)],
                           inf16, mask=lane0)
        up = (plsc.bitcast(u, jnp.int32) & MASK8) | iota
        up = jnp.where(iota == cstar, pinf16, up)
        nm = jnp.min(up)
        plsc.store_scatter(aux, [ghi, glo],
                           jnp.full((16,), nm, jnp.int32), mask=lane0)
        p = mstar + 48 * iota
        w = plsc.load_gather(aux, [lax.shift_right_logical(p, 7), p & 127])
        wl = jnp.where(iota == l2, nm, w) | (iota * 16)
        mb2 = jnp.min(wl)
        plsc.store_scatter(aux, [jnp.full((16,), 6, jnp.int32),
                                 jnp.full((16,), mstar, jnp.int32)],
                           jnp.full((16,), mb2, jnp.int32), mask=lane0)
        return nn_lo, nn_hi

    def process2(r0, bp):
        rowA = rowbuf.at[bp]
        auxA = auxbuf.at[bp]
        rowB = rowbuf.at[bp + 1]
        auxB = auxbuf.at[bp + 1]

        def ext2(it, c2):
            alo, ahi, blo, bhi = c2
            alo, ahi = one_ext(rowA, auxA, it, alo, ahi)
            blo, bhi = one_ext(rowB, auxB, it, blo, bhi)
            return (alo, ahi, blo, bhi)
        zi = jnp.zeros((16,), jnp.int32)
        alo, ahi, blo, bhi = lax.fori_loop(0, K, ext2, (zi, zi, zi, zi))
        nnbuf[0, pl.ds(0, 16)] = alo
        nnbuf[0, pl.ds(16, 16)] = ahi
        nnbuf[1, pl.ds(0, 16)] = blo
        nnbuf[1, pl.ds(16, 16)] = bhi

        pltpu.sync_copy(nnbuf.at[0], nn_hbm.at[r0])

        @pl.when(r0 + 1 < hi)
        def _():
            pltpu.sync_copy(nnbuf.at[1], nn_hbm.at[r0 + 1])

    startp(base, 0, semr0, sema0)

    def quad_step(qq, carry):
        r0 = base + 4 * qq

        @pl.when(r0 + 2 < hi)
        def _():
            startp(r0 + 2, 2, semr1, sema1)
        waitp(r0, 0, semr0, sema0)
        process2(r0, 0)

        @pl.when(r0 + 2 < hi)
        def _():
            @pl.when(r0 + 4 < hi)
            def _():
                startp(r0 + 4, 0, semr0, sema0)
            waitp(r0 + 2, 2, semr1, sema1)
            process2(r0 + 2, 2)
        return carry

    lax.fori_loop(0, (cnt + 3) // 4, quad_step, 0)


def _select(d3, aux):
    mesh = plsc.VectorSubcoreMesh(core_axis_name="c", subcore_axis_name="s")
    return pl.kernel(
        _select_body,
        out_type=jax.ShapeDtypeStruct((NNR, K), jnp.int32),
        mesh=mesh,
        compiler_params=pltpu.CompilerParams(needs_layout_passes=False),
        scratch_types=[pltpu.VMEM((4, 16, 640), jnp.float32),
                       pltpu.VMEM((4, 8, 128), jnp.int32),
                       pltpu.VMEM((2, K), jnp.int32),
                       pltpu.SemaphoreType.DMA,
                       pltpu.SemaphoreType.DMA,
                       pltpu.SemaphoreType.DMA,
                       pltpu.SemaphoreType.DMA],
    )(d3, aux)


def _layer_body(xt_hbm, nn_hbm, zz_hbm, out_hbm, idxall, rows, ftE, accum,
                semg):
    cid = lax.axis_index("c")
    sid = lax.axis_index("s")
    wid = sid * 2 + cid
    base = wid * RPW
    cnt = jnp.minimum(RPW, N - base)
    iota = lax.iota(jnp.int32, 16)
    pltpu.sync_copy(zz_hbm.at[pl.ds(sid * EPT, EPT)],
                    accum.at[pl.ds(sid * EPT, EPT)])
    pltpu.sync_copy(nn_hbm.at[pl.ds(base, RPW)], idxall)
    cpad = jnp.where(iota == 0, jnp.float32(1.0), jnp.float32(0.0))
    for j in range(K):
        ftE[j, pl.ds(128, 16)] = cpad
    plsc.subcore_barrier()

    def start(ee, b):
        pltpu.async_copy(xt_hbm.at[idxall.at[ee]], rows.at[b], semg)

    def wait(b):
        pltpu.make_async_copy(xt_hbm.at[idxall.at[0]], rows.at[b],
                              semg).wait()

    def process(ee, b):
        accs = [jnp.zeros((16,), jnp.float32) for _ in range(8)]
        for j in range(K):
            for t in range(8):
                accs[t] = accs[t] + rows[b, j, pl.ds(t * 16, 16)]
        scale = jnp.float32(1.0 / K)
        accs = [a * scale for a in accs]
        for j in range(K):
            for t in range(8):
                ftE[j, pl.ds(t * 16, 16)] = accs[t]
        pltpu.sync_copy(ftE, accum.at[idxall.at[ee]], add=True)

    start(0, 0)

    def pair_step(q, carry):
        e0 = 2 * q

        @pl.when(e0 + 1 < cnt)
        def _():
            start(e0 + 1, 1)
        wait(0)
        process(e0, 0)

        @pl.when(e0 + 2 < cnt)
        def _():
            start(e0 + 2, 0)

        @pl.when(e0 + 1 < cnt)
        def _():
            wait(1)
            process(e0 + 1, 1)
        return carry

    lax.fori_loop(0, (cnt + 1) // 2, pair_step, 0)
    plsc.subcore_barrier()
    pltpu.sync_copy(accum.at[pl.ds(sid * EPT, EPT)],
                    out_hbm.at[cid, pl.ds(sid * EPT, EPT)])


def _layer(xt, nn, zz):
    mesh = plsc.VectorSubcoreMesh(core_axis_name="c", subcore_axis_name="s")
    return pl.kernel(
        _layer_body,
        out_type=jax.ShapeDtypeStruct((2, N, AUGW), jnp.float32),
        mesh=mesh,
        compiler_params=pltpu.CompilerParams(needs_layout_passes=False,
                                             use_tc_tiling_on_sc=False),
        scratch_types=[pltpu.VMEM((RPW, K), jnp.int32),
                       pltpu.VMEM((2, K, C), jnp.float32),
                       pltpu.VMEM((K, AUGW), jnp.float32),
                       pltpu.VMEM_SHARED((N, AUGW), jnp.float32),
                       pltpu.SemaphoreType.DMA],
    )(xt, nn, zz)


def _mm_body(x_ref, w_ref, o_ref):
    o_ref[...] = jnp.dot(x_ref[...], w_ref[...],
                         preferred_element_type=jnp.float32)


def _mm(x, w):
    RB = 2000
    return pl.pallas_call(
        _mm_body,
        grid=(N // RB,),
        in_specs=[pl.BlockSpec((RB, C), lambda i: (i, 0)),
                  pl.BlockSpec((C, C), lambda i: (0, 0))],
        out_specs=pl.BlockSpec((RB, C), lambda i: (i, 0)),
        out_shape=jax.ShapeDtypeStruct((N, C), jnp.float32),
    )(x, w)


def _mid_body(p_ref, b_ref, w_ref, o_ref):
    p = p_ref[...]
    s = p[0] + p[1]
    dv = jnp.maximum(s[:, 128:129], 1.0)
    h = s[:, 0:128] / dv + b_ref[...]
    h = jnp.where(h >= 0, h, 0.01 * h)
    o_ref[...] = jnp.dot(h, w_ref[...], preferred_element_type=jnp.float32)


def _mid(p, bias, w):
    RB = 2000
    return pl.pallas_call(
        _mid_body,
        grid=(N // RB,),
        in_specs=[pl.BlockSpec((2, RB, AUGW), lambda i: (0, i, 0)),
                  pl.BlockSpec((1, C), lambda i: (0, 0)),
                  pl.BlockSpec((C, C), lambda i: (0, 0))],
        out_specs=pl.BlockSpec((RB, C), lambda i: (i, 0)),
        out_shape=jax.ShapeDtypeStruct((N, C), jnp.float32),
    )(p, bias, w)


def _fin_body(p_ref, b_ref, wfc_ref, bfc_ref, feats_ref, pool_ref, out_ref,
              acc_ref):
    i = pl.program_id(0)
    p = p_ref[...]
    s = p[0] + p[1]
    dv = jnp.maximum(s[:, 128:129], 1.0)
    h = s[:, 0:128] / dv + b_ref[...]
    h = jnp.where(h >= 0, h, 0.01 * h)
    feats_ref[...] = h

    @pl.when(i == 0)
    def _():
        acc_ref[...] = jnp.zeros_like(acc_ref)

    acc_ref[...] += jnp.sum(h, axis=0, keepdims=True)

    @pl.when(i == pl.num_programs(0) - 1)
    def _():
        pool = acc_ref[...] * jnp.float32(1.0 / N)
        pool_ref[...] = pool
        z = jnp.dot(pool, wfc_ref[...], preferred_element_type=jnp.float32)
        out_ref[...] = jax.nn.sigmoid(z + bfc_ref[...])


def _fin(p, bias, wfc, bfc):
    RB = 2000
    return pl.pallas_call(
        _fin_body,
        grid=(N // RB,),
        in_specs=[pl.BlockSpec((2, RB, AUGW), lambda i: (0, i, 0)),
                  pl.BlockSpec((1, C), lambda i: (0, 0)),
                  pl.BlockSpec((C, 2), lambda i: (0, 0)),
                  pl.BlockSpec((1, 2), lambda i: (0, 0))],
        out_specs=[pl.BlockSpec((RB, C), lambda i: (i, 0)),
                   pl.BlockSpec((1, C), lambda i: (0, 0)),
                   pl.BlockSpec((1, 2), lambda i: (0, 0))],
        out_shape=[jax.ShapeDtypeStruct((N, C), jnp.float32),
                   jax.ShapeDtypeStruct((1, C), jnp.float32),
                   jax.ShapeDtypeStruct((1, 2), jnp.float32)],
        scratch_shapes=[pltpu.VMEM((1, C), jnp.float32)],
    )(p, bias, wfc, bfc)


def kernel(x, theta0, bias0, theta1, bias1, W_fc, b_fc):
    xpad = jnp.pad(x, ((0, NP - N), (0, 0)))
    d3, aux = _build_dist(xpad, xpad.T)
    nn = _select(d3, aux)
    zz = jnp.zeros((N, AUGW), jnp.float32)
    x1 = _mm(x, theta0)
    p1 = _layer(x1, nn, zz)
    x2 = _mid(p1, bias0.reshape(1, C), theta1)
    p2 = _layer(x2, nn, zz)
    feats, pool, out2 = _fin(p2, bias1.reshape(1, C), W_fc,
                             b_fc.reshape(1, 2))
    return (out2[0], feats, pool)


# speedup vs baseline: 1.0597x; 1.0597x over previous
import jax
import jax.numpy as jnp
from jax import lax
from jax.experimental import pallas as pl
from jax.experimental.pallas import tpu as pltpu
from jax.experimental.pallas import tpu_sc as plsc

N = 10000
C = 128
K = 32
NP = 10240
AUGW = 144
NW = 32
RPW = 313
EPT = 625
NNR = NW * RPW
INF = 3.0e38
PINF = 0x7F61B1E6
MASK8 = -256


def _dist_body(xr_ref, xt_ref, out_ref, aux_ref):
    a = xr_ref[...]
    sa = jnp.sum(a * a, axis=1, keepdims=True)
    mm = None
    for l in range(16):
        b = xt_ref[:, pl.ds(l * 640, 640)]
        sb = jnp.sum(b * b, axis=0, keepdims=True)
        g = jnp.dot(a, b, preferred_element_type=jnp.float32)
        d = sa + sb - 2.0 * g
        if l == 15:
            col = lax.broadcasted_iota(jnp.int32, d.shape, 1)
            d = jnp.where(col + 9600 < N, d, INF)
        out_ref[:, l, :] = d
        dp = (lax.bitcast_convert_type(d, jnp.int32) & MASK8) | l
        mm = dp if mm is None else jnp.minimum(mm, dp)
    rb = a.shape[0]
    pinf128 = jnp.full((rb, 128), PINF, jnp.int32)
    for t in range(5):
        aux_ref[:, t, :] = mm[:, t * 128:(t + 1) * 128]
    aux_ref[:, 5, :] = pinf128
    aux_ref[:, 7, :] = pinf128
    mm768 = jnp.concatenate([mm, pinf128], axis=1)
    mb = None
    for l in range(16):
        w = mm768[:, l * 48:(l + 1) * 48] | (l << 4)
        mb = w if mb is None else jnp.minimum(mb, w)
    aux_ref[:, 6, :] = jnp.concatenate(
        [mb, jnp.full((rb, 80), PINF, jnp.int32)], axis=1)


def _build_dist(xpad, xpadT):
    RB = 128
    return pl.pallas_call(
        _dist_body,
        grid=(NP // RB,),
        in_specs=[pl.BlockSpec((RB, C), lambda i: (i, 0)),
                  pl.BlockSpec((C, NP), lambda i: (0, 0))],
        out_specs=[pl.BlockSpec((RB, 16, 640), lambda i: (i, 0, 0)),
                   pl.BlockSpec((RB, 8, 128), lambda i: (i, 0, 0))],
        out_shape=[jax.ShapeDtypeStruct((NP, 16, 640), jnp.float32),
                   jax.ShapeDtypeStruct((NP, 8, 128), jnp.int32)],
    )(xpad, xpadT)


def _select_body(d_hbm, aux_hbm, nn_hbm, rowbuf, auxbuf, nnbuf,
                 semr0, sema0, semr1, sema1):
    cid = lax.axis_index("c")
    sid = lax.axis_index("s")
    wid = sid * 2 + cid
    base = wid * RPW
    cnt = jnp.minimum(RPW, N - base)
    hi = base + cnt
    iota = lax.iota(jnp.int32, 16)
    big = jnp.int32(9999)
    inf16 = jnp.full((16,), INF, jnp.float32)
    pinf16 = jnp.full((16,), PINF, jnp.int32)
    lane0 = iota == 0

    def startp(r0, bp, sr, sa2):
        pltpu.async_copy(d_hbm.at[r0], rowbuf.at[bp], sr)
        pltpu.async_copy(aux_hbm.at[r0], auxbuf.at[bp], sa2)

        @pl.when(r0 + 1 < hi)
        def _():
            pltpu.async_copy(d_hbm.at[r0 + 1], rowbuf.at[bp + 1], sr)
            pltpu.async_copy(aux_hbm.at[r0 + 1], auxbuf.at[bp + 1], sa2)

    def waitp(r0, bp, sr, sa2):
        pltpu.make_async_copy(d_hbm.at[base], rowbuf.at[bp], sr).wait()
        pltpu.make_async_copy(aux_hbm.at[base], auxbuf.at[bp], sa2).wait()

        @pl.when(r0 + 1 < hi)
        def _():
            pltpu.make_async_copy(d_hbm.at[base], rowbuf.at[bp + 1],
                                  sr).wait()
            pltpu.make_async_copy(aux_hbm.at[base], auxbuf.at[bp + 1],
                                  sa2).wait()

    def one_ext(row, aux, it, nn_lo, nn_hi):
        v0 = aux[6, pl.ds(0, 16)]
        v1 = aux[6, pl.ds(16, 16)]
        v2 = aux[6, pl.ds(32, 16)]
        gmin = jnp.min(jnp.minimum(jnp.minimum(v0, v1), v2))
        m0 = plsc.all_reduce_ffs(v0 == gmin)
        m1 = plsc.all_reduce_ffs(v1 == gmin)
        m2 = plsc.all_reduce_ffs(v2 == gmin)
        msv = jnp.where(m0 < 16, m0, jnp.where(m1 < 16, m1 + 16, m2 + 32))
        mstar = msv[0]
        l2 = lax.shift_right_logical(gmin, 4) & 0xF
        g = mstar + 48 * l2
        cstar = gmin & 0xF
        ghi = jnp.full((16,), lax.shift_right_logical(g, 7), jnp.int32)
        glo = jnp.full((16,), g & 127, jnp.int32)
        u = plsc.load_gather(row, [iota, jnp.full((16,), g, jnp.int32)])
        colidx = cstar * 640 + g
        nn_lo = jnp.where(iota == it, colidx, nn_lo)
        nn_hi = jnp.where(iota == it - 16, colidx, nn_hi)
        plsc.store_scatter(row, [jnp.full((16,), cstar, jnp.int32),
                                 jnp.full((16,), g, jnp.int32)],
                           inf16, mask=lane0)
        up = (plsc.bitcast(u, jnp.int32) & MASK8) | iota
        up = jnp.where(iota == cstar, pinf16, up)
        nm = jnp.min(up)
        plsc.store_scatter(aux, [ghi, glo],
                           jnp.full((16,), nm, jnp.int32), mask=lane0)
        p = mstar + 48 * iota
        w = plsc.load_gather(aux, [lax.shift_right_logical(p, 7), p & 127])
        wl = jnp.where(iota == l2, nm, w) | (iota * 16)
        mb2 = jnp.min(wl)
        plsc.store_scatter(aux, [jnp.full((16,), 6, jnp.int32),
                                 jnp.full((16,), mstar, jnp.int32)],
                           jnp.full((16,), mb2, jnp.int32), mask=lane0)
        return nn_lo, nn_hi

    def process2(r0, bp):
        rowA = rowbuf.at[bp]
        auxA = auxbuf.at[bp]
        rowB = rowbuf.at[bp + 1]
        auxB = auxbuf.at[bp + 1]

        def ext2(it, c2):
            alo, ahi, blo, bhi = c2
            alo, ahi = one_ext(rowA, auxA, it, alo, ahi)
            blo, bhi = one_ext(rowB, auxB, it, blo, bhi)
            return (alo, ahi, blo, bhi)
        zi = jnp.zeros((16,), jnp.int32)
        alo, ahi, blo, bhi = lax.fori_loop(0, K, ext2, (zi, zi, zi, zi))
        nnbuf[0, pl.ds(0, 16)] = alo
        nnbuf[0, pl.ds(16, 16)] = ahi
        nnbuf[1, pl.ds(0, 16)] = blo
        nnbuf[1, pl.ds(16, 16)] = bhi

        pltpu.sync_copy(nnbuf.at[0], nn_hbm.at[r0])

        @pl.when(r0 + 1 < hi)
        def _():
            pltpu.sync_copy(nnbuf.at[1], nn_hbm.at[r0 + 1])

    startp(base, 0, semr0, sema0)

    def quad_step(qq, carry):
        r0 = base + 4 * qq

        @pl.when(r0 + 2 < hi)
        def _():
            startp(r0 + 2, 2, semr1, sema1)
        waitp(r0, 0, semr0, sema0)
        process2(r0, 0)

        @pl.when(r0 + 2 < hi)
        def _():
            @pl.when(r0 + 4 < hi)
            def _():
                startp(r0 + 4, 0, semr0, sema0)
            waitp(r0 + 2, 2, semr1, sema1)
            process2(r0 + 2, 2)
        return carry

    lax.fori_loop(0, (cnt + 3) // 4, quad_step, 0)


def _select(d3, aux):
    mesh = plsc.VectorSubcoreMesh(core_axis_name="c", subcore_axis_name="s")
    return pl.kernel(
        _select_body,
        out_type=jax.ShapeDtypeStruct((NNR, K), jnp.int32),
        mesh=mesh,
        compiler_params=pltpu.CompilerParams(needs_layout_passes=False),
        scratch_types=[pltpu.VMEM((4, 16, 640), jnp.float32),
                       pltpu.VMEM((4, 8, 128), jnp.int32),
                       pltpu.VMEM((2, K), jnp.int32),
                       pltpu.SemaphoreType.DMA,
                       pltpu.SemaphoreType.DMA,
                       pltpu.SemaphoreType.DMA,
                       pltpu.SemaphoreType.DMA],
    )(d3, aux)


def _layer_body(xt_hbm, nn_hbm, zz_hbm, out_hbm, idxall, rows, ftE, accum,
                semg, sems0, sems1):
    cid = lax.axis_index("c")
    sid = lax.axis_index("s")
    wid = sid * 2 + cid
    base = wid * RPW
    cnt = jnp.minimum(RPW, N - base)
    iota = lax.iota(jnp.int32, 16)
    pltpu.sync_copy(zz_hbm.at[pl.ds(sid * EPT, EPT)],
                    accum.at[pl.ds(sid * EPT, EPT)])
    pltpu.sync_copy(nn_hbm.at[pl.ds(base, RPW)], idxall)
    cpad = jnp.where(iota == 0, jnp.float32(1.0), jnp.float32(0.0))
    for bf in range(2):
        for j in range(K):
            ftE[bf, j, pl.ds(128, 16)] = cpad
    plsc.subcore_barrier()
    scale = jnp.float32(1.0 / K)

    def start(ee, b):
        pltpu.async_copy(xt_hbm.at[idxall.at[ee]], rows.at[b], semg)

    def waitg(b):
        pltpu.make_async_copy(xt_hbm.at[idxall.at[0]], rows.at[b],
                              semg).wait()

    def waits(bf, sem_s):
        pltpu.make_async_copy(ftE.at[bf], accum.at[idxall.at[0]],
                              sem_s).wait()

    def slot(e, br, sem_s, prev):
        bf = 1 - br

        @pl.when(e >= 3)
        def _():
            waits(bf, sem_s)
        accs = [jnp.zeros((16,), jnp.float32) for _ in range(8)]
        for j in range(K):
            for t in range(8):
                accs[t] = accs[t] + rows[br, j, pl.ds(t * 16, 16)]
                ftE[bf, j, pl.ds(t * 16, 16)] = prev[t]

        @pl.when(e >= 1)
        def _():
            pltpu.async_copy(ftE.at[bf], accum.at[idxall.at[e - 1]], sem_s,
                             add=True)
        return tuple(a * scale for a in accs)

    start(0, 0)

    def pair_step(q, prev):
        e0 = 2 * q

        @pl.when(e0 + 1 < cnt)
        def _():
            start(e0 + 1, 1)
        waitg(0)
        prev = slot(e0, 0, sems1, prev)

        @pl.when(e0 + 2 < cnt)
        def _():
            start(e0 + 2, 0)

        @pl.when(e0 + 1 < cnt)
        def _():
            waitg(1)
        prev = slot(e0 + 1, 1, sems0, prev)
        return prev

    zv = tuple(jnp.zeros((16,), jnp.float32) for _ in range(8))
    prev = lax.fori_loop(0, (cnt + 1) // 2, pair_step, zv)

    @pl.when(cnt % 2 == 0)
    def _():
        waits(1, sems1)
        for j in range(K):
            for t in range(8):
                ftE[1, j, pl.ds(t * 16, 16)] = prev[t]
        pltpu.sync_copy(ftE.at[1], accum.at[idxall.at[cnt - 1]], add=True)

    waits(0, sems0)

    @pl.when(cnt % 2 == 1)
    def _():
        waits(1, sems1)
    plsc.subcore_barrier()
    pltpu.sync_copy(accum.at[pl.ds(sid * EPT, EPT)],
                    out_hbm.at[cid, pl.ds(sid * EPT, EPT)])


def _layer(xt, nn, zz):
    mesh = plsc.VectorSubcoreMesh(core_axis_name="c", subcore_axis_name="s")
    return pl.kernel(
        _layer_body,
        out_type=jax.ShapeDtypeStruct((2, N, AUGW), jnp.float32),
        mesh=mesh,
        compiler_params=pltpu.CompilerParams(needs_layout_passes=False,
                                             use_tc_tiling_on_sc=False),
        scratch_types=[pltpu.VMEM((RPW, K), jnp.int32),
                       pltpu.VMEM((2, K, C), jnp.float32),
                       pltpu.VMEM((2, K, AUGW), jnp.float32),
                       pltpu.VMEM_SHARED((N, AUGW), jnp.float32),
                       pltpu.SemaphoreType.DMA,
                       pltpu.SemaphoreType.DMA,
                       pltpu.SemaphoreType.DMA],
    )(xt, nn, zz)


def _mm_body(x_ref, w_ref, o_ref):
    o_ref[...] = jnp.dot(x_ref[...], w_ref[...],
                         preferred_element_type=jnp.float32)


def _mm(x, w):
    RB = 2000
    return pl.pallas_call(
        _mm_body,
        grid=(N // RB,),
        in_specs=[pl.BlockSpec((RB, C), lambda i: (i, 0)),
                  pl.BlockSpec((C, C), lambda i: (0, 0))],
        out_specs=pl.BlockSpec((RB, C), lambda i: (i, 0)),
        out_shape=jax.ShapeDtypeStruct((N, C), jnp.float32),
    )(x, w)


def _mid_body(p_ref, b_ref, w_ref, o_ref):
    p = p_ref[...]
    s = p[0] + p[1]
    dv = jnp.maximum(s[:, 128:129], 1.0)
    h = s[:, 0:128] / dv + b_ref[...]
    h = jnp.where(h >= 0, h, 0.01 * h)
    o_ref[...] = jnp.dot(h, w_ref[...], preferred_element_type=jnp.float32)


def _mid(p, bias, w):
    RB = 2000
    return pl.pallas_call(
        _mid_body,
        grid=(N // RB,),
        in_specs=[pl.BlockSpec((2, RB, AUGW), lambda i: (0, i, 0)),
                  pl.BlockSpec((1, C), lambda i: (0, 0)),
                  pl.BlockSpec((C, C), lambda i: (0, 0))],
        out_specs=pl.BlockSpec((RB, C), lambda i: (i, 0)),
        out_shape=jax.ShapeDtypeStruct((N, C), jnp.float32),
    )(p, bias, w)


def _fin_body(p_ref, b_ref, wfc_ref, bfc_ref, feats_ref, pool_ref, out_ref,
              acc_ref):
    i = pl.program_id(0)
    p = p_ref[...]
    s = p[0] + p[1]
    dv = jnp.maximum(s[:, 128:129], 1.0)
    h = s[:, 0:128] / dv + b_ref[...]
    h = jnp.where(h >= 0, h, 0.01 * h)
    feats_ref[...] = h

    @pl.when(i == 0)
    def _():
        acc_ref[...] = jnp.zeros_like(acc_ref)

    acc_ref[...] += jnp.sum(h, axis=0, keepdims=True)

    @pl.when(i == pl.num_programs(0) - 1)
    def _():
        pool = acc_ref[...] * jnp.float32(1.0 / N)
        pool_ref[...] = pool
        z = jnp.dot(pool, wfc_ref[...], preferred_element_type=jnp.float32)
        out_ref[...] = jax.nn.sigmoid(z + bfc_ref[...])


def _fin(p, bias, wfc, bfc):
    RB = 2000
    return pl.pallas_call(
        _fin_body,
        grid=(N // RB,),
        in_specs=[pl.BlockSpec((2, RB, AUGW), lambda i: (0, i, 0)),
                  pl.BlockSpec((1, C), lambda i: (0, 0)),
                  pl.BlockSpec((C, 2), lambda i: (0, 0)),
                  pl.BlockSpec((1, 2), lambda i: (0, 0))],
        out_specs=[pl.BlockSpec((RB, C), lambda i: (i, 0)),
                   pl.BlockSpec((1, C), lambda i: (0, 0)),
                   pl.BlockSpec((1, 2), lambda i: (0, 0))],
        out_shape=[jax.ShapeDtypeStruct((N, C), jnp.float32),
                   jax.ShapeDtypeStruct((1, C), jnp.float32),
                   jax.ShapeDtypeStruct((1, 2), jnp.float32)],
        scratch_shapes=[pltpu.VMEM((1, C), jnp.float32)],
    )(p, bias, wfc, bfc)


def kernel(x, theta0, bias0, theta1, bias1, W_fc, b_fc):
    xpad = jnp.pad(x, ((0, NP - N), (0, 0)))
    d3, aux = _build_dist(xpad, xpad.T)
    nn = _select(d3, aux)
    zz = jnp.zeros((N, AUGW), jnp.float32)
    x1 = _mm(x, theta0)
    p1 = _layer(x1, nn, zz)
    x2 = _mid(p1, bias0.reshape(1, C), theta1)
    p2 = _layer(x2, nn, zz)
    feats, pool, out2 = _fin(p2, bias1.reshape(1, C), W_fc,
                             b_fc.reshape(1, 2))
    return (out2[0], feats, pool)


# speedup vs baseline: 1.1870x; 1.1201x over previous
import jax
import jax.numpy as jnp
from jax import lax
from jax.experimental import pallas as pl
from jax.experimental.pallas import tpu as pltpu
from jax.experimental.pallas import tpu_sc as plsc

N = 10000
C = 128
K = 32
NP = 10240
AUGW = 144
NW = 32
RPW = 313
EPT = 625
NNR = NW * RPW
INF = 3.0e38
PINF = 0x7F61B1E6
MASK8 = -256


def _dist_body(xr_ref, xt_ref, out_ref, aux_ref):
    a = xr_ref[...]
    sa = jnp.sum(a * a, axis=1, keepdims=True)
    mm = None
    for l in range(16):
        b = xt_ref[:, pl.ds(l * 640, 640)]
        sb = jnp.sum(b * b, axis=0, keepdims=True)
        g = jnp.dot(a, b, preferred_element_type=jnp.float32)
        d = sa + sb - 2.0 * g
        if l == 15:
            col = lax.broadcasted_iota(jnp.int32, d.shape, 1)
            d = jnp.where(col + 9600 < N, d, INF)
        out_ref[:, l, :] = d
        dp = (lax.bitcast_convert_type(d, jnp.int32) & MASK8) | l
        mm = dp if mm is None else jnp.minimum(mm, dp)
    rb = a.shape[0]
    pinf128 = jnp.full((rb, 128), PINF, jnp.int32)
    for t in range(5):
        aux_ref[:, t, :] = mm[:, t * 128:(t + 1) * 128]
    aux_ref[:, 5, :] = pinf128
    aux_ref[:, 7, :] = pinf128
    mm768 = jnp.concatenate([mm, pinf128], axis=1)
    mb = None
    for l in range(16):
        w = mm768[:, l * 48:(l + 1) * 48] | (l << 4)
        mb = w if mb is None else jnp.minimum(mb, w)
    aux_ref[:, 6, :] = jnp.concatenate(
        [mb, jnp.full((rb, 80), PINF, jnp.int32)], axis=1)


def _build_dist(xpad, xpadT, nrows):
    RB = 128
    return pl.pallas_call(
        _dist_body,
        grid=(nrows // RB,),
        in_specs=[pl.BlockSpec((RB, C), lambda i: (i, 0)),
                  pl.BlockSpec((C, NP), lambda i: (0, 0))],
        out_specs=[pl.BlockSpec((RB, 16, 640), lambda i: (i, 0, 0)),
                   pl.BlockSpec((RB, 8, 128), lambda i: (i, 0, 0))],
        out_shape=[jax.ShapeDtypeStruct((nrows, 16, 640), jnp.float32),
                   jax.ShapeDtypeStruct((nrows, 8, 128), jnp.int32)],
    )(xpad, xpadT)


def _make_select_body(nrows, rpw):
  def _select_body(d_hbm, aux_hbm, nn_hbm, rowbuf, auxbuf, nnbuf,
                   semr0, sema0, semr1, sema1):
    cid = lax.axis_index("c")
    sid = lax.axis_index("s")
    wid = sid * 2 + cid
    base = wid * rpw
    cnt = jnp.minimum(rpw, nrows - base)
    hi = base + cnt
    iota = lax.iota(jnp.int32, 16)
    big = jnp.int32(9999)
    inf16 = jnp.full((16,), INF, jnp.float32)
    pinf16 = jnp.full((16,), PINF, jnp.int32)
    lane0 = iota == 0

    def startp(r0, bp, sr, sa2):
        pltpu.async_copy(d_hbm.at[r0], rowbuf.at[bp], sr)
        pltpu.async_copy(aux_hbm.at[r0], auxbuf.at[bp], sa2)

        @pl.when(r0 + 1 < hi)
        def _():
            pltpu.async_copy(d_hbm.at[r0 + 1], rowbuf.at[bp + 1], sr)
            pltpu.async_copy(aux_hbm.at[r0 + 1], auxbuf.at[bp + 1], sa2)

    def waitp(r0, bp, sr, sa2):
        pltpu.make_async_copy(d_hbm.at[base], rowbuf.at[bp], sr).wait()
        pltpu.make_async_copy(aux_hbm.at[base], auxbuf.at[bp], sa2).wait()

        @pl.when(r0 + 1 < hi)
        def _():
            pltpu.make_async_copy(d_hbm.at[base], rowbuf.at[bp + 1],
                                  sr).wait()
            pltpu.make_async_copy(aux_hbm.at[base], auxbuf.at[bp + 1],
                                  sa2).wait()

    def one_ext(row, aux, it, nn_lo, nn_hi):
        v0 = aux[6, pl.ds(0, 16)]
        v1 = aux[6, pl.ds(16, 16)]
        v2 = aux[6, pl.ds(32, 16)]
        gmin = jnp.min(jnp.minimum(jnp.minimum(v0, v1), v2))
        m0 = plsc.all_reduce_ffs(v0 == gmin)
        m1 = plsc.all_reduce_ffs(v1 == gmin)
        m2 = plsc.all_reduce_ffs(v2 == gmin)
        msv = jnp.where(m0 < 16, m0, jnp.where(m1 < 16, m1 + 16, m2 + 32))
        mstar = msv[0]
        l2 = lax.shift_right_logical(gmin, 4) & 0xF
        g = mstar + 48 * l2
        cstar = gmin & 0xF
        ghi = jnp.full((16,), lax.shift_right_logical(g, 7), jnp.int32)
        glo = jnp.full((16,), g & 127, jnp.int32)
        u = plsc.load_gather(row, [iota, jnp.full((16,), g, jnp.int32)])
        colidx = cstar * 640 + g
        nn_lo = jnp.where(iota == it, colidx, nn_lo)
        nn_hi = jnp.where(iota == it - 16, colidx, nn_hi)
        plsc.store_scatter(row, [jnp.full((16,), cstar, jnp.int32),
                                 jnp.full((16,), g, jnp.int32)],
                           inf16, mask=lane0)
        up = (plsc.bitcast(u, jnp.int32) & MASK8) | iota
        up = jnp.where(iota == cstar, pinf16, up)
        nm = jnp.min(up)
        plsc.store_scatter(aux, [ghi, glo],
                           jnp.full((16,), nm, jnp.int32), mask=lane0)
        p = mstar + 48 * iota
        w = plsc.load_gather(aux, [lax.shift_right_logical(p, 7), p & 127])
        wl = jnp.where(iota == l2, nm, w) | (iota * 16)
        mb2 = jnp.min(wl)
        plsc.store_scatter(aux, [jnp.full((16,), 6, jnp.int32),
                                 jnp.full((16,), mstar, jnp.int32)],
                           jnp.full((16,), mb2, jnp.int32), mask=lane0)
        return nn_lo, nn_hi

    def process2(r0, bp):
        rowA = rowbuf.at[bp]
        auxA = auxbuf.at[bp]
        rowB = rowbuf.at[bp + 1]
        auxB = auxbuf.at[bp + 1]

        def ext2(it, c2):
            alo, ahi, blo, bhi = c2
            alo, ahi = one_ext(rowA, auxA, it, alo, ahi)
            blo, bhi = one_ext(rowB, auxB, it, blo, bhi)
            return (alo, ahi, blo, bhi)
        zi = jnp.zeros((16,), jnp.int32)
        alo, ahi, blo, bhi = lax.fori_loop(0, K, ext2, (zi, zi, zi, zi))
        nnbuf[0, pl.ds(0, 16)] = alo
        nnbuf[0, pl.ds(16, 16)] = ahi
        nnbuf[1, pl.ds(0, 16)] = blo
        nnbuf[1, pl.ds(16, 16)] = bhi

        pltpu.sync_copy(nnbuf.at[0], nn_hbm.at[r0])

        @pl.when(r0 + 1 < hi)
        def _():
            pltpu.sync_copy(nnbuf.at[1], nn_hbm.at[r0 + 1])

    startp(base, 0, semr0, sema0)

    def quad_step(qq, carry):
        r0 = base + 4 * qq

        @pl.when(r0 + 2 < hi)
        def _():
            startp(r0 + 2, 2, semr1, sema1)
        waitp(r0, 0, semr0, sema0)
        process2(r0, 0)

        @pl.when(r0 + 2 < hi)
        def _():
            @pl.when(r0 + 4 < hi)
            def _():
                startp(r0 + 4, 0, semr0, sema0)
            waitp(r0 + 2, 2, semr1, sema1)
            process2(r0 + 2, 2)
        return carry

    lax.fori_loop(0, (cnt + 3) // 4, quad_step, 0)
  return _select_body


def _select(d3, aux, nrows, rpw):
    mesh = plsc.VectorSubcoreMesh(core_axis_name="c", subcore_axis_name="s")
    return pl.kernel(
        _make_select_body(nrows, rpw),
        out_type=jax.ShapeDtypeStruct((NW * rpw, K), jnp.int32),
        mesh=mesh,
        compiler_params=pltpu.CompilerParams(needs_layout_passes=False),
        scratch_types=[pltpu.VMEM((4, 16, 640), jnp.float32),
                       pltpu.VMEM((4, 8, 128), jnp.int32),
                       pltpu.VMEM((2, K), jnp.int32),
                       pltpu.SemaphoreType.DMA,
                       pltpu.SemaphoreType.DMA,
                       pltpu.SemaphoreType.DMA,
                       pltpu.SemaphoreType.DMA],
    )(d3, aux)


def _layer_body(xt_hbm, nn_hbm, zz_hbm, out_hbm, idxall, rows, ftE, accum,
                semg, sems0, sems1):
    cid = lax.axis_index("c")
    sid = lax.axis_index("s")
    wid = sid * 2 + cid
    base = wid * RPW
    cnt = jnp.minimum(RPW, N - base)
    iota = lax.iota(jnp.int32, 16)
    pltpu.sync_copy(zz_hbm.at[pl.ds(sid * EPT, EPT)],
                    accum.at[pl.ds(sid * EPT, EPT)])
    pltpu.sync_copy(nn_hbm.at[pl.ds(base, RPW)], idxall)
    cpad = jnp.where(iota == 0, jnp.float32(1.0), jnp.float32(0.0))
    for bf in range(2):
        for j in range(K):
            ftE[bf, j, pl.ds(128, 16)] = cpad
    plsc.subcore_barrier()
    scale = jnp.float32(1.0 / K)

    def start(ee, b):
        pltpu.async_copy(xt_hbm.at[idxall.at[ee]], rows.at[b], semg)

    def waitg(b):
        pltpu.make_async_copy(xt_hbm.at[idxall.at[0]], rows.at[b],
                              semg).wait()

    def waits(bf, sem_s):
        pltpu.make_async_copy(ftE.at[bf], accum.at[idxall.at[0]],
                              sem_s).wait()

    def slot(e, br, sem_s, prev):
        bf = 1 - br

        @pl.when(e >= 3)
        def _():
            waits(bf, sem_s)
        accs = [jnp.zeros((16,), jnp.float32) for _ in range(8)]
        for j in range(K):
            for t in range(8):
                accs[t] = accs[t] + rows[br, j, pl.ds(t * 16, 16)]
                ftE[bf, j, pl.ds(t * 16, 16)] = prev[t]

        @pl.when(e >= 1)
        def _():
            pltpu.async_copy(ftE.at[bf], accum.at[idxall.at[e - 1]], sem_s,
                             add=True)
        return tuple(a * scale for a in accs)

    start(0, 0)

    def pair_step(q, prev):
        e0 = 2 * q

        @pl.when(e0 + 1 < cnt)
        def _():
            start(e0 + 1, 1)
        waitg(0)
        prev = slot(e0, 0, sems1, prev)

        @pl.when(e0 + 2 < cnt)
        def _():
            start(e0 + 2, 0)

        @pl.when(e0 + 1 < cnt)
        def _():
            waitg(1)
        prev = slot(e0 + 1, 1, sems0, prev)
        return prev

    zv = tuple(jnp.zeros((16,), jnp.float32) for _ in range(8))
    prev = lax.fori_loop(0, (cnt + 1) // 2, pair_step, zv)

    @pl.when(cnt % 2 == 0)
    def _():
        waits(1, sems1)
        for j in range(K):
            for t in range(8):
                ftE[1, j, pl.ds(t * 16, 16)] = prev[t]
        pltpu.sync_copy(ftE.at[1], accum.at[idxall.at[cnt - 1]], add=True)

    waits(0, sems0)

    @pl.when(cnt % 2 == 1)
    def _():
        waits(1, sems1)
    plsc.subcore_barrier()
    pltpu.sync_copy(accum.at[pl.ds(sid * EPT, EPT)],
                    out_hbm.at[cid, pl.ds(sid * EPT, EPT)])


def _layer(xt, nn, zz):
    mesh = plsc.VectorSubcoreMesh(core_axis_name="c", subcore_axis_name="s")
    return pl.kernel(
        _layer_body,
        out_type=jax.ShapeDtypeStruct((2, N, AUGW), jnp.float32),
        mesh=mesh,
        compiler_params=pltpu.CompilerParams(needs_layout_passes=False,
                                             use_tc_tiling_on_sc=False),
        scratch_types=[pltpu.VMEM((RPW, K), jnp.int32),
                       pltpu.VMEM((2, K, C), jnp.float32),
                       pltpu.VMEM((2, K, AUGW), jnp.float32),
                       pltpu.VMEM_SHARED((N, AUGW), jnp.float32),
                       pltpu.SemaphoreType.DMA,
                       pltpu.SemaphoreType.DMA,
                       pltpu.SemaphoreType.DMA],
    )(xt, nn, zz)


def _mm_body(x_ref, w_ref, o_ref):
    o_ref[...] = jnp.dot(x_ref[...], w_ref[...],
                         preferred_element_type=jnp.float32)


def _mm(x, w):
    RB = 2000
    return pl.pallas_call(
        _mm_body,
        grid=(N // RB,),
        in_specs=[pl.BlockSpec((RB, C), lambda i: (i, 0)),
                  pl.BlockSpec((C, C), lambda i: (0, 0))],
        out_specs=pl.BlockSpec((RB, C), lambda i: (i, 0)),
        out_shape=jax.ShapeDtypeStruct((N, C), jnp.float32),
    )(x, w)


def _mid_body(p_ref, b_ref, w_ref, o_ref):
    p = p_ref[...]
    s = p[0] + p[1]
    dv = jnp.maximum(s[:, 128:129], 1.0)
    h = s[:, 0:128] / dv + b_ref[...]
    h = jnp.where(h >= 0, h, 0.01 * h)
    o_ref[...] = jnp.dot(h, w_ref[...], preferred_element_type=jnp.float32)


def _mid(p, bias, w):
    RB = 2000
    return pl.pallas_call(
        _mid_body,
        grid=(N // RB,),
        in_specs=[pl.BlockSpec((2, RB, AUGW), lambda i: (0, i, 0)),
                  pl.BlockSpec((1, C), lambda i: (0, 0)),
                  pl.BlockSpec((C, C), lambda i: (0, 0))],
        out_specs=pl.BlockSpec((RB, C), lambda i: (i, 0)),
        out_shape=jax.ShapeDtypeStruct((N, C), jnp.float32),
    )(p, bias, w)


def _fin_body(p_ref, b_ref, wfc_ref, bfc_ref, feats_ref, pool_ref, out_ref,
              acc_ref):
    i = pl.program_id(0)
    p = p_ref[...]
    s = p[0] + p[1]
    dv = jnp.maximum(s[:, 128:129], 1.0)
    h = s[:, 0:128] / dv + b_ref[...]
    h = jnp.where(h >= 0, h, 0.01 * h)
    feats_ref[...] = h

    @pl.when(i == 0)
    def _():
        acc_ref[...] = jnp.zeros_like(acc_ref)

    acc_ref[...] += jnp.sum(h, axis=0, keepdims=True)

    @pl.when(i == pl.num_programs(0) - 1)
    def _():
        pool = acc_ref[...] * jnp.float32(1.0 / N)
        pool_ref[...] = pool
        z = jnp.dot(pool, wfc_ref[...], preferred_element_type=jnp.float32)
        out_ref[...] = jax.nn.sigmoid(z + bfc_ref[...])


def _fin(p, bias, wfc, bfc):
    RB = 2000
    return pl.pallas_call(
        _fin_body,
        grid=(N // RB,),
        in_specs=[pl.BlockSpec((2, RB, AUGW), lambda i: (0, i, 0)),
                  pl.BlockSpec((1, C), lambda i: (0, 0)),
                  pl.BlockSpec((C, 2), lambda i: (0, 0)),
                  pl.BlockSpec((1, 2), lambda i: (0, 0))],
        out_specs=[pl.BlockSpec((RB, C), lambda i: (i, 0)),
                   pl.BlockSpec((1, C), lambda i: (0, 0)),
                   pl.BlockSpec((1, 2), lambda i: (0, 0))],
        out_shape=[jax.ShapeDtypeStruct((N, C), jnp.float32),
                   jax.ShapeDtypeStruct((1, C), jnp.float32),
                   jax.ShapeDtypeStruct((1, 2), jnp.float32)],
        scratch_shapes=[pltpu.VMEM((1, C), jnp.float32)],
    )(p, bias, wfc, bfc)


def kernel(x, theta0, bias0, theta1, bias1, W_fc, b_fc):
    xpad = jnp.pad(x, ((0, NP - N), (0, 0)))
    xpadT = xpad.T
    da, auxa = _build_dist(xpad[:5120], xpadT, 5120)
    db, auxb = _build_dist(xpad[5120:], xpadT, 5120)
    nna = _select(da, auxa, 5120, 160)
    nnb = _select(db, auxb, 4880, 153)
    nn = jnp.concatenate([nna, nnb], axis=0)
    zz = jnp.zeros((N, AUGW), jnp.float32)
    x1 = _mm(x, theta0)
    p1 = _layer(x1, nn, zz)
    x2 = _mid(p1, bias0.reshape(1, C), theta1)
    p2 = _layer(x2, nn, zz)
    feats, pool, out2 = _fin(p2, bias1.reshape(1, C), W_fc,
                             b_fc.reshape(1, 2))
    return (out2[0], feats, pool)


# speedup vs baseline: 1.4636x; 1.2330x over previous
import jax
import jax.numpy as jnp
from jax import lax
from jax.experimental import pallas as pl
from jax.experimental.pallas import tpu as pltpu
from jax.experimental.pallas import tpu_sc as plsc

N = 10000
C = 128
K = 32
NP = 10240
AUGW = 144
NW = 32
RPW = 313
EPT = 625
NNR = NW * RPW
INF = 3.0e38
PINF = 0x7F61B1E6
MASK8 = -256


def _dist_body(xr_ref, xt_ref, out_ref, aux_ref):
    a = xr_ref[...]
    sa = jnp.sum(a * a, axis=1, keepdims=True)
    mm = None
    for l in range(16):
        b = xt_ref[:, pl.ds(l * 640, 640)]
        sb = jnp.sum(b * b, axis=0, keepdims=True)
        g = jnp.dot(a, b, preferred_element_type=jnp.float32)
        d = sa + sb - 2.0 * g
        if l == 15:
            col = lax.broadcasted_iota(jnp.int32, d.shape, 1)
            d = jnp.where(col + 9600 < N, d, INF)
        out_ref[:, l, :] = d
        mm = d if mm is None else jnp.minimum(mm, d)
    rb = a.shape[0]
    pinf128 = jnp.full((rb, 128), INF, jnp.float32)
    for t in range(5):
        aux_ref[:, t, :] = mm[:, t * 128:(t + 1) * 128]
    aux_ref[:, 5, :] = pinf128
    aux_ref[:, 7, :] = pinf128
    mm768 = jnp.concatenate([mm, pinf128], axis=1)
    mb = None
    for l in range(16):
        w = mm768[:, l * 48:(l + 1) * 48]
        mb = w if mb is None else jnp.minimum(mb, w)
    aux_ref[:, 6, :] = jnp.concatenate(
        [mb, jnp.full((rb, 80), INF, jnp.float32)], axis=1)


def _build_dist(xpad, xpadT, nrows):
    RB = 128
    return pl.pallas_call(
        _dist_body,
        grid=(nrows // RB,),
        in_specs=[pl.BlockSpec((RB, C), lambda i: (i, 0)),
                  pl.BlockSpec((C, NP), lambda i: (0, 0))],
        out_specs=[pl.BlockSpec((RB, 16, 640), lambda i: (i, 0, 0)),
                   pl.BlockSpec((RB, 8, 128), lambda i: (i, 0, 0))],
        out_shape=[jax.ShapeDtypeStruct((nrows, 16, 640), jnp.float32),
                   jax.ShapeDtypeStruct((nrows, 8, 128), jnp.float32)],
    )(xpad, xpadT)


def _make_select_body(nrows, rpw):
  def _select_body(d_hbm, aux_hbm, nn_hbm, rowbuf, auxbuf, nnbuf,
                   semr0, sema0, semr1, sema1):
    cid = lax.axis_index("c")
    sid = lax.axis_index("s")
    wid = sid * 2 + cid
    base = wid * rpw
    cnt = jnp.minimum(rpw, nrows - base)
    hi = base + cnt
    iota = lax.iota(jnp.int32, 16)
    big = jnp.int32(9999)
    inf16 = jnp.full((16,), INF, jnp.float32)
    pinf16 = jnp.full((16,), PINF, jnp.int32)
    lane0 = iota == 0

    def startp(r0, bp, sr, sa2):
        pltpu.async_copy(d_hbm.at[r0], rowbuf.at[bp], sr)
        pltpu.async_copy(aux_hbm.at[r0], auxbuf.at[bp], sa2)

        @pl.when(r0 + 1 < hi)
        def _():
            pltpu.async_copy(d_hbm.at[r0 + 1], rowbuf.at[bp + 1], sr)
            pltpu.async_copy(aux_hbm.at[r0 + 1], auxbuf.at[bp + 1], sa2)

    def waitp(r0, bp, sr, sa2):
        pltpu.make_async_copy(d_hbm.at[base], rowbuf.at[bp], sr).wait()
        pltpu.make_async_copy(aux_hbm.at[base], auxbuf.at[bp], sa2).wait()

        @pl.when(r0 + 1 < hi)
        def _():
            pltpu.make_async_copy(d_hbm.at[base], rowbuf.at[bp + 1],
                                  sr).wait()
            pltpu.make_async_copy(aux_hbm.at[base], auxbuf.at[bp + 1],
                                  sa2).wait()

    def one_ext(row, aux, it, nn_lo, nn_hi):
        v0 = aux[6, pl.ds(0, 16)]
        v1 = aux[6, pl.ds(16, 16)]
        v2 = aux[6, pl.ds(32, 16)]
        gmin = jnp.min(jnp.minimum(jnp.minimum(v0, v1), v2))
        m0 = plsc.all_reduce_ffs(v0 == gmin)
        m1 = plsc.all_reduce_ffs(v1 == gmin)
        m2 = plsc.all_reduce_ffs(v2 == gmin)
        msv = jnp.where(m0 < 16, m0, jnp.where(m1 < 16, m1 + 16, m2 + 32))
        mstar = msv[0]
        p = msv + 48 * iota
        w = plsc.load_gather(aux, [lax.shift_right_logical(p, 7), p & 127])
        lsv = plsc.all_reduce_ffs(w == gmin)
        lstar = lsv[0]
        g = mstar + 48 * lstar
        ghi = jnp.full((16,), lax.shift_right_logical(g, 7), jnp.int32)
        glo = jnp.full((16,), g & 127, jnp.int32)
        u = plsc.load_gather(row, [iota, jnp.full((16,), g, jnp.int32)])
        csv = plsc.all_reduce_ffs(u == gmin)
        cstar = csv[0]
        colidx = cstar * 640 + g
        nn_lo = jnp.where(iota == it, colidx, nn_lo)
        nn_hi = jnp.where(iota == it - 16, colidx, nn_hi)
        plsc.store_scatter(row, [jnp.full((16,), cstar, jnp.int32),
                                 jnp.full((16,), g, jnp.int32)],
                           inf16, mask=lane0)
        nm = jnp.min(jnp.where(iota == cstar, inf16, u))
        plsc.store_scatter(aux, [ghi, glo],
                           jnp.full((16,), nm, jnp.float32), mask=lane0)
        mb2 = jnp.min(jnp.where(iota == lstar, nm, w))
        plsc.store_scatter(aux, [jnp.full((16,), 6, jnp.int32),
                                 jnp.full((16,), mstar, jnp.int32)],
                           jnp.full((16,), mb2, jnp.float32), mask=lane0)
        return nn_lo, nn_hi

    def process2(r0, bp):
        rowA = rowbuf.at[bp]
        auxA = auxbuf.at[bp]
        rowB = rowbuf.at[bp + 1]
        auxB = auxbuf.at[bp + 1]

        def ext2(it, c2):
            alo, ahi, blo, bhi = c2
            alo, ahi = one_ext(rowA, auxA, it, alo, ahi)
            blo, bhi = one_ext(rowB, auxB, it, blo, bhi)
            return (alo, ahi, blo, bhi)
        zi = jnp.zeros((16,), jnp.int32)
        alo, ahi, blo, bhi = lax.fori_loop(0, K, ext2, (zi, zi, zi, zi))
        nnbuf[0, pl.ds(0, 16)] = alo
        nnbuf[0, pl.ds(16, 16)] = ahi
        nnbuf[1, pl.ds(0, 16)] = blo
        nnbuf[1, pl.ds(16, 16)] = bhi

        pltpu.sync_copy(nnbuf.at[0], nn_hbm.at[r0])

        @pl.when(r0 + 1 < hi)
        def _():
            pltpu.sync_copy(nnbuf.at[1], nn_hbm.at[r0 + 1])

    startp(base, 0, semr0, sema0)

    def quad_step(qq, carry):
        r0 = base + 4 * qq

        @pl.when(r0 + 2 < hi)
        def _():
            startp(r0 + 2, 2, semr1, sema1)
        waitp(r0, 0, semr0, sema0)
        process2(r0, 0)

        @pl.when(r0 + 2 < hi)
        def _():
            @pl.when(r0 + 4 < hi)
            def _():
                startp(r0 + 4, 0, semr0, sema0)
            waitp(r0 + 2, 2, semr1, sema1)
            process2(r0 + 2, 2)
        return carry

    lax.fori_loop(0, (cnt + 3) // 4, quad_step, 0)
  return _select_body


def _select(d3, aux, nrows, rpw):
    mesh = plsc.VectorSubcoreMesh(core_axis_name="c", subcore_axis_name="s")
    return pl.kernel(
        _make_select_body(nrows, rpw),
        out_type=jax.ShapeDtypeStruct((NW * rpw, K), jnp.int32),
        mesh=mesh,
        compiler_params=pltpu.CompilerParams(needs_layout_passes=False),
        scratch_types=[pltpu.VMEM((4, 16, 640), jnp.float32),
                       pltpu.VMEM((4, 8, 128), jnp.float32),
                       pltpu.VMEM((2, K), jnp.int32),
                       pltpu.SemaphoreType.DMA,
                       pltpu.SemaphoreType.DMA,
                       pltpu.SemaphoreType.DMA,
                       pltpu.SemaphoreType.DMA],
    )(d3, aux)


def _layer_body(xt_hbm, nn_hbm, zz_hbm, out_hbm, idxall, rows, ftE, accum,
                semg, sems0, sems1):
    cid = lax.axis_index("c")
    sid = lax.axis_index("s")
    wid = sid * 2 + cid
    base = wid * RPW
    cnt = jnp.minimum(RPW, N - base)
    iota = lax.iota(jnp.int32, 16)
    pltpu.sync_copy(zz_hbm.at[pl.ds(sid * EPT, EPT)],
                    accum.at[pl.ds(sid * EPT, EPT)])
    pltpu.sync_copy(nn_hbm.at[pl.ds(base, RPW)], idxall)
    cpad = jnp.where(iota == 0, jnp.float32(1.0), jnp.float32(0.0))
    for bf in range(2):
        for j in range(K):
            ftE[bf, j, pl.ds(128, 16)] = cpad
    plsc.subcore_barrier()
    scale = jnp.float32(1.0 / K)

    def start(ee, b):
        pltpu.async_copy(xt_hbm.at[idxall.at[ee]], rows.at[b], semg)

    def waitg(b):
        pltpu.make_async_copy(xt_hbm.at[idxall.at[0]], rows.at[b],
                              semg).wait()

    def waits(bf, sem_s):
        pltpu.make_async_copy(ftE.at[bf], accum.at[idxall.at[0]],
                              sem_s).wait()

    def slot(e, br, sem_s, prev):
        bf = 1 - br

        @pl.when(e >= 3)
        def _():
            waits(bf, sem_s)
        accs = [jnp.zeros((16,), jnp.float32) for _ in range(8)]
        for j in range(K):
            for t in range(8):
                accs[t] = accs[t] + rows[br, j, pl.ds(t * 16, 16)]
                ftE[bf, j, pl.ds(t * 16, 16)] = prev[t]

        @pl.when(e >= 1)
        def _():
            pltpu.async_copy(ftE.at[bf], accum.at[idxall.at[e - 1]], sem_s,
                             add=True)
        return tuple(a * scale for a in accs)

    start(0, 0)

    def pair_step(q, prev):
        e0 = 2 * q

        @pl.when(e0 + 1 < cnt)
        def _():
            start(e0 + 1, 1)
        waitg(0)
        prev = slot(e0, 0, sems1, prev)

        @pl.when(e0 + 2 < cnt)
        def _():
            start(e0 + 2, 0)

        @pl.when(e0 + 1 < cnt)
        def _():
            waitg(1)
        prev = slot(e0 + 1, 1, sems0, prev)
        return prev

    zv = tuple(jnp.zeros((16,), jnp.float32) for _ in range(8))
    prev = lax.fori_loop(0, (cnt + 1) // 2, pair_step, zv)

    @pl.when(cnt % 2 == 0)
    def _():
        waits(1, sems1)
        for j in range(K):
            for t in range(8):
                ftE[1, j, pl.ds(t * 16, 16)] = prev[t]
        pltpu.sync_copy(ftE.at[1], accum.at[idxall.at[cnt - 1]], add=True)

    waits(0, sems0)

    @pl.when(cnt % 2 == 1)
    def _():
        waits(1, sems1)
    plsc.subcore_barrier()
    pltpu.sync_copy(accum.at[pl.ds(sid * EPT, EPT)],
                    out_hbm.at[cid, pl.ds(sid * EPT, EPT)])


def _layer(xt, nn, zz):
    mesh = plsc.VectorSubcoreMesh(core_axis_name="c", subcore_axis_name="s")
    return pl.kernel(
        _layer_body,
        out_type=jax.ShapeDtypeStruct((2, N, AUGW), jnp.float32),
        mesh=mesh,
        compiler_params=pltpu.CompilerParams(needs_layout_passes=False,
                                             use_tc_tiling_on_sc=False),
        scratch_types=[pltpu.VMEM((RPW, K), jnp.int32),
                       pltpu.VMEM((2, K, C), jnp.float32),
                       pltpu.VMEM((2, K, AUGW), jnp.float32),
                       pltpu.VMEM_SHARED((N, AUGW), jnp.float32),
                       pltpu.SemaphoreType.DMA,
                       pltpu.SemaphoreType.DMA,
                       pltpu.SemaphoreType.DMA],
    )(xt, nn, zz)


def _mm_body(x_ref, w_ref, o_ref):
    o_ref[...] = jnp.dot(x_ref[...], w_ref[...],
                         preferred_element_type=jnp.float32)


def _mm(x, w):
    RB = 2000
    return pl.pallas_call(
        _mm_body,
        grid=(N // RB,),
        in_specs=[pl.BlockSpec((RB, C), lambda i: (i, 0)),
                  pl.BlockSpec((C, C), lambda i: (0, 0))],
        out_specs=pl.BlockSpec((RB, C), lambda i: (i, 0)),
        out_shape=jax.ShapeDtypeStruct((N, C), jnp.float32),
    )(x, w)


def _mid_body(p_ref, b_ref, w_ref, o_ref):
    p = p_ref[...]
    s = p[0] + p[1]
    dv = jnp.maximum(s[:, 128:129], 1.0)
    h = s[:, 0:128] / dv + b_ref[...]
    h = jnp.where(h >= 0, h, 0.01 * h)
    o_ref[...] = jnp.dot(h, w_ref[...], preferred_element_type=jnp.float32)


def _mid(p, bias, w):
    RB = 2000
    return pl.pallas_call(
        _mid_body,
        grid=(N // RB,),
        in_specs=[pl.BlockSpec((2, RB, AUGW), lambda i: (0, i, 0)),
                  pl.BlockSpec((1, C), lambda i: (0, 0)),
                  pl.BlockSpec((C, C), lambda i: (0, 0))],
        out_specs=pl.BlockSpec((RB, C), lambda i: (i, 0)),
        out_shape=jax.ShapeDtypeStruct((N, C), jnp.float32),
    )(p, bias, w)


def _fin_body(p_ref, b_ref, wfc_ref, bfc_ref, feats_ref, pool_ref, out_ref,
              acc_ref):
    i = pl.program_id(0)
    p = p_ref[...]
    s = p[0] + p[1]
    dv = jnp.maximum(s[:, 128:129], 1.0)
    h = s[:, 0:128] / dv + b_ref[...]
    h = jnp.where(h >= 0, h, 0.01 * h)
    feats_ref[...] = h

    @pl.when(i == 0)
    def _():
        acc_ref[...] = jnp.zeros_like(acc_ref)

    acc_ref[...] += jnp.sum(h, axis=0, keepdims=True)

    @pl.when(i == pl.num_programs(0) - 1)
    def _():
        pool = acc_ref[...] * jnp.float32(1.0 / N)
        pool_ref[...] = pool
        z = jnp.dot(pool, wfc_ref[...], preferred_element_type=jnp.float32)
        out_ref[...] = jax.nn.sigmoid(z + bfc_ref[...])


def _fin(p, bias, wfc, bfc):
    RB = 2000
    return pl.pallas_call(
        _fin_body,
        grid=(N // RB,),
        in_specs=[pl.BlockSpec((2, RB, AUGW), lambda i: (0, i, 0)),
                  pl.BlockSpec((1, C), lambda i: (0, 0)),
                  pl.BlockSpec((C, 2), lambda i: (0, 0)),
                  pl.BlockSpec((1, 2), lambda i: (0, 0))],
        out_specs=[pl.BlockSpec((RB, C), lambda i: (i, 0)),
                   pl.BlockSpec((1, C), lambda i: (0, 0)),
                   pl.BlockSpec((1, 2), lambda i: (0, 0))],
        out_shape=[jax.ShapeDtypeStruct((N, C), jnp.float32),
                   jax.ShapeDtypeStruct((1, C), jnp.float32),
                   jax.ShapeDtypeStruct((1, 2), jnp.float32)],
        scratch_shapes=[pltpu.VMEM((1, C), jnp.float32)],
    )(p, bias, wfc, bfc)


def kernel(x, theta0, bias0, theta1, bias1, W_fc, b_fc):
    xpad = jnp.pad(x, ((0, NP - N), (0, 0)))
    xpadT = xpad.T
    parts = []
    for lo, nr, rows_valid, rpw in ((0, 2560, 2560, 80),
                                    (2560, 2560, 2560, 80),
                                    (5120, 2560, 2560, 80),
                                    (7680, 2560, 2320, 73)):
        dq, auxq = _build_dist(xpad[lo:lo + nr], xpadT, nr)
        parts.append(_select(dq, auxq, rows_valid, rpw))
    nn = jnp.concatenate(parts, axis=0)
    zz = jnp.zeros((N, AUGW), jnp.float32)
    x1 = _mm(x, theta0)
    p1 = _layer(x1, nn, zz)
    x2 = _mid(p1, bias0.reshape(1, C), theta1)
    p2 = _layer(x2, nn, zz)
    feats, pool, out2 = _fin(p2, bias1.reshape(1, C), W_fc,
                             b_fc.reshape(1, 2))
    return (out2[0], feats, pool)


# speedup vs baseline: 1.4647x; 1.0008x over previous
import jax
import jax.numpy as jnp
from jax import lax
from jax.experimental import pallas as pl
from jax.experimental.pallas import tpu as pltpu
from jax.experimental.pallas import tpu_sc as plsc

N = 10000
C = 128
K = 32
NP = 10240
AUGW = 144
NW = 32
RPW = 313
EPT = 625
NNR = NW * RPW
INF = 3.0e38


def _dist_body(xr_ref, xt_ref, out_ref, aux_ref):
    a = xr_ref[...]
    sa = jnp.sum(a * a, axis=1, keepdims=True)
    mm = None
    for l in range(16):
        b = xt_ref[:, pl.ds(l * 640, 640)]
        sb = jnp.sum(b * b, axis=0, keepdims=True)
        g = jnp.dot(a, b, preferred_element_type=jnp.float32)
        d = sa + sb - 2.0 * g
        if l == 15:
            col = lax.broadcasted_iota(jnp.int32, d.shape, 1)
            d = jnp.where(col + 9600 < N, d, INF)
        out_ref[:, l, :] = d
        mm = d if mm is None else jnp.minimum(mm, d)
    rb = a.shape[0]
    pinf128 = jnp.full((rb, 128), INF, jnp.float32)
    for t in range(5):
        aux_ref[:, t, :] = mm[:, t * 128:(t + 1) * 128]
    aux_ref[:, 5, :] = pinf128
    aux_ref[:, 7, :] = pinf128
    mm768 = jnp.concatenate([mm, pinf128], axis=1)
    mb = None
    for l in range(16):
        w = mm768[:, l * 48:(l + 1) * 48]
        mb = w if mb is None else jnp.minimum(mb, w)
    aux_ref[:, 6, :] = jnp.concatenate(
        [mb, jnp.full((rb, 80), INF, jnp.float32)], axis=1)


def _build_dist(xpad, xpadT, nrows):
    RB = 128
    return pl.pallas_call(
        _dist_body,
        grid=(nrows // RB,),
        in_specs=[pl.BlockSpec((RB, C), lambda i: (i, 0)),
                  pl.BlockSpec((C, NP), lambda i: (0, 0))],
        out_specs=[pl.BlockSpec((RB, 16, 640), lambda i: (i, 0, 0)),
                   pl.BlockSpec((RB, 8, 128), lambda i: (i, 0, 0))],
        out_shape=[jax.ShapeDtypeStruct((nrows, 16, 640), jnp.float32),
                   jax.ShapeDtypeStruct((nrows, 8, 128), jnp.float32)],
    )(xpad, xpadT)


def _make_select_body(nrows, rpw):
  def _select_body(d_hbm, aux_hbm, nn_hbm, rowbuf, auxbuf, nnbuf,
                   semr0, sema0, semr1, sema1):
    cid = lax.axis_index("c")
    sid = lax.axis_index("s")
    wid = sid * 2 + cid
    base = wid * rpw
    cnt = jnp.minimum(rpw, nrows - base)
    hi = base + cnt
    iota = lax.iota(jnp.int32, 16)
    big = jnp.int32(9999)
    inf16 = jnp.full((16,), INF, jnp.float32)
    lane0 = iota == 0

    def startp(r0, bp, sr, sa2):
        pltpu.async_copy(d_hbm.at[r0], rowbuf.at[bp], sr)
        pltpu.async_copy(aux_hbm.at[r0], auxbuf.at[bp], sa2)

        @pl.when(r0 + 1 < hi)
        def _():
            pltpu.async_copy(d_hbm.at[r0 + 1], rowbuf.at[bp + 1], sr)
            pltpu.async_copy(aux_hbm.at[r0 + 1], auxbuf.at[bp + 1], sa2)

    def waitp(r0, bp, sr, sa2):
        pltpu.make_async_copy(d_hbm.at[base], rowbuf.at[bp], sr).wait()
        pltpu.make_async_copy(aux_hbm.at[base], auxbuf.at[bp], sa2).wait()

        @pl.when(r0 + 1 < hi)
        def _():
            pltpu.make_async_copy(d_hbm.at[base], rowbuf.at[bp + 1],
                                  sr).wait()
            pltpu.make_async_copy(aux_hbm.at[base], auxbuf.at[bp + 1],
                                  sa2).wait()

    def one_ext(row, aux, it, nn_lo, nn_hi):
        v0 = aux[6, pl.ds(0, 16)]
        v1 = aux[6, pl.ds(16, 16)]
        v2 = aux[6, pl.ds(32, 16)]
        gmin = jnp.min(jnp.minimum(jnp.minimum(v0, v1), v2))
        m0 = plsc.all_reduce_ffs(v0 == gmin)
        m1 = plsc.all_reduce_ffs(v1 == gmin)
        m2 = plsc.all_reduce_ffs(v2 == gmin)
        msv = jnp.where(m0 < 16, m0, jnp.where(m1 < 16, m1 + 16, m2 + 32))
        mstar = msv[0]
        p = msv + 48 * iota
        w = plsc.load_gather(aux, [lax.shift_right_logical(p, 7), p & 127])
        lsv = plsc.all_reduce_ffs(w == gmin)
        lstar = lsv[0]
        g = mstar + 48 * lstar
        ghi = jnp.full((16,), lax.shift_right_logical(g, 7), jnp.int32)
        glo = jnp.full((16,), g & 127, jnp.int32)
        u = plsc.load_gather(row, [iota, jnp.full((16,), g, jnp.int32)])
        csv = plsc.all_reduce_ffs(u == gmin)
        cstar = csv[0]
        colidx = cstar * 640 + g
        nn_lo = jnp.where(iota == it, colidx, nn_lo)
        nn_hi = jnp.where(iota == it - 16, colidx, nn_hi)
        plsc.store_scatter(row, [jnp.full((16,), cstar, jnp.int32),
                                 jnp.full((16,), g, jnp.int32)],
                           inf16, mask=lane0)
        nm = jnp.min(jnp.where(iota == cstar, inf16, u))
        plsc.store_scatter(aux, [ghi, glo],
                           jnp.full((16,), nm, jnp.float32), mask=lane0)
        mb2 = jnp.min(jnp.where(iota == lstar, nm, w))
        plsc.store_scatter(aux, [jnp.full((16,), 6, jnp.int32),
                                 jnp.full((16,), mstar, jnp.int32)],
                           jnp.full((16,), mb2, jnp.float32), mask=lane0)
        return nn_lo, nn_hi

    def process2(r0, bp):
        rowA = rowbuf.at[bp]
        auxA = auxbuf.at[bp]
        rowB = rowbuf.at[bp + 1]
        auxB = auxbuf.at[bp + 1]

        def ext2(it, c2):
            alo, ahi, blo, bhi = c2
            alo, ahi = one_ext(rowA, auxA, it, alo, ahi)
            blo, bhi = one_ext(rowB, auxB, it, blo, bhi)
            return (alo, ahi, blo, bhi)
        zi = jnp.zeros((16,), jnp.int32)
        alo, ahi, blo, bhi = lax.fori_loop(0, K, ext2, (zi, zi, zi, zi))
        nnbuf[0, pl.ds(0, 16)] = alo
        nnbuf[0, pl.ds(16, 16)] = ahi
        nnbuf[1, pl.ds(0, 16)] = blo
        nnbuf[1, pl.ds(16, 16)] = bhi

        pltpu.sync_copy(nnbuf.at[0], nn_hbm.at[r0])

        @pl.when(r0 + 1 < hi)
        def _():
            pltpu.sync_copy(nnbuf.at[1], nn_hbm.at[r0 + 1])

    startp(base, 0, semr0, sema0)

    def quad_step(qq, carry):
        r0 = base + 4 * qq

        @pl.when(r0 + 2 < hi)
        def _():
            startp(r0 + 2, 2, semr1, sema1)
        waitp(r0, 0, semr0, sema0)
        process2(r0, 0)

        @pl.when(r0 + 2 < hi)
        def _():
            @pl.when(r0 + 4 < hi)
            def _():
                startp(r0 + 4, 0, semr0, sema0)
            waitp(r0 + 2, 2, semr1, sema1)
            process2(r0 + 2, 2)
        return carry

    lax.fori_loop(0, (cnt + 3) // 4, quad_step, 0)
  return _select_body


def _select(d3, aux, nrows, rpw):
    mesh = plsc.VectorSubcoreMesh(core_axis_name="c", subcore_axis_name="s")
    return pl.kernel(
        _make_select_body(nrows, rpw),
        out_type=jax.ShapeDtypeStruct((NW * rpw, K), jnp.int32),
        mesh=mesh,
        compiler_params=pltpu.CompilerParams(needs_layout_passes=False),
        scratch_types=[pltpu.VMEM((4, 16, 640), jnp.float32),
                       pltpu.VMEM((4, 8, 128), jnp.float32),
                       pltpu.VMEM((2, K), jnp.int32),
                       pltpu.SemaphoreType.DMA,
                       pltpu.SemaphoreType.DMA,
                       pltpu.SemaphoreType.DMA,
                       pltpu.SemaphoreType.DMA],
    )(d3, aux)


def _layer_body(xt_hbm, nn_hbm, zz_hbm, out_hbm, idxall, rows, ftE, accum,
                semg, sems0, sems1):
    cid = lax.axis_index("c")
    sid = lax.axis_index("s")
    wid = sid * 2 + cid
    base = wid * RPW
    cnt = jnp.minimum(RPW, N - base)
    iota = lax.iota(jnp.int32, 16)
    pltpu.sync_copy(zz_hbm.at[pl.ds(sid * EPT, EPT)],
                    accum.at[pl.ds(sid * EPT, EPT)])
    pltpu.sync_copy(nn_hbm.at[pl.ds(base, RPW)], idxall)
    cpad = jnp.where(iota == 0, jnp.float32(1.0), jnp.float32(0.0))
    for bf in range(2):
        for j in range(K):
            ftE[bf, j, pl.ds(128, 16)] = cpad
    plsc.subcore_barrier()
    scale = jnp.float32(1.0 / K)

    def start(ee, b):
        pltpu.async_copy(xt_hbm.at[idxall.at[ee]], rows.at[b], semg)

    def waitg(b):
        pltpu.make_async_copy(xt_hbm.at[idxall.at[0]], rows.at[b],
                              semg).wait()

    def waits(bf, sem_s):
        pltpu.make_async_copy(ftE.at[bf], accum.at[idxall.at[0]],
                              sem_s).wait()

    def slot(e, br, sem_s, prev):
        bf = 1 - br

        @pl.when(e >= 3)
        def _():
            waits(bf, sem_s)
        accs = [jnp.zeros((16,), jnp.float32) for _ in range(8)]
        for j in range(K):
            for t in range(8):
                accs[t] = accs[t] + rows[br, j, pl.ds(t * 16, 16)]
                ftE[bf, j, pl.ds(t * 16, 16)] = prev[t]

        @pl.when(e >= 1)
        def _():
            pltpu.async_copy(ftE.at[bf], accum.at[idxall.at[e - 1]], sem_s,
                             add=True)
        return tuple(a * scale for a in accs)

    start(0, 0)

    def pair_step(q, prev):
        e0 = 2 * q

        @pl.when(e0 + 1 < cnt)
        def _():
            start(e0 + 1, 1)
        waitg(0)
        prev = slot(e0, 0, sems1, prev)

        @pl.when(e0 + 2 < cnt)
        def _():
            start(e0 + 2, 0)

        @pl.when(e0 + 1 < cnt)
        def _():
            waitg(1)
        prev = slot(e0 + 1, 1, sems0, prev)
        return prev

    zv = tuple(jnp.zeros((16,), jnp.float32) for _ in range(8))
    prev = lax.fori_loop(0, (cnt + 1) // 2, pair_step, zv)

    @pl.when(cnt % 2 == 0)
    def _():
        waits(1, sems1)
        for j in range(K):
            for t in range(8):
                ftE[1, j, pl.ds(t * 16, 16)] = prev[t]
        pltpu.sync_copy(ftE.at[1], accum.at[idxall.at[cnt - 1]], add=True)

    waits(0, sems0)

    @pl.when(cnt % 2 == 1)
    def _():
        waits(1, sems1)
    plsc.subcore_barrier()
    pltpu.sync_copy(accum.at[pl.ds(sid * EPT, EPT)],
                    out_hbm.at[cid, pl.ds(sid * EPT, EPT)])


def _layer(xt, nn, zz):
    mesh = plsc.VectorSubcoreMesh(core_axis_name="c", subcore_axis_name="s")
    return pl.kernel(
        _layer_body,
        out_type=jax.ShapeDtypeStruct((2, N, AUGW), jnp.float32),
        mesh=mesh,
        compiler_params=pltpu.CompilerParams(needs_layout_passes=False,
                                             use_tc_tiling_on_sc=False),
        scratch_types=[pltpu.VMEM((RPW, K), jnp.int32),
                       pltpu.VMEM((2, K, C), jnp.float32),
                       pltpu.VMEM((2, K, AUGW), jnp.float32),
                       pltpu.VMEM_SHARED((N, AUGW), jnp.float32),
                       pltpu.SemaphoreType.DMA,
                       pltpu.SemaphoreType.DMA,
                       pltpu.SemaphoreType.DMA],
    )(xt, nn, zz)


def _mm_body(x_ref, w_ref, o_ref):
    o_ref[...] = jnp.dot(x_ref[...], w_ref[...],
                         preferred_element_type=jnp.float32)


def _mm(x, w):
    RB = 2000
    return pl.pallas_call(
        _mm_body,
        grid=(N // RB,),
        in_specs=[pl.BlockSpec((RB, C), lambda i: (i, 0)),
                  pl.BlockSpec((C, C), lambda i: (0, 0))],
        out_specs=pl.BlockSpec((RB, C), lambda i: (i, 0)),
        out_shape=jax.ShapeDtypeStruct((N, C), jnp.float32),
    )(x, w)


def _mid_body(p_ref, b_ref, w_ref, o_ref):
    p = p_ref[...]
    s = p[0] + p[1]
    dv = jnp.maximum(s[:, 128:129], 1.0)
    h = s[:, 0:128] / dv + b_ref[...]
    h = jnp.where(h >= 0, h, 0.01 * h)
    o_ref[...] = jnp.dot(h, w_ref[...], preferred_element_type=jnp.float32)


def _mid(p, bias, w):
    RB = 2000
    return pl.pallas_call(
        _mid_body,
        grid=(N // RB,),
        in_specs=[pl.BlockSpec((2, RB, AUGW), lambda i: (0, i, 0)),
                  pl.BlockSpec((1, C), lambda i: (0, 0)),
                  pl.BlockSpec((C, C), lambda i: (0, 0))],
        out_specs=pl.BlockSpec((RB, C), lambda i: (i, 0)),
        out_shape=jax.ShapeDtypeStruct((N, C), jnp.float32),
    )(p, bias, w)


def _fin_body(p_ref, b_ref, wfc_ref, bfc_ref, feats_ref, pool_ref, out_ref,
              acc_ref):
    i = pl.program_id(0)
    p = p_ref[...]
    s = p[0] + p[1]
    dv = jnp.maximum(s[:, 128:129], 1.0)
    h = s[:, 0:128] / dv + b_ref[...]
    h = jnp.where(h >= 0, h, 0.01 * h)
    feats_ref[...] = h

    @pl.when(i == 0)
    def _():
        acc_ref[...] = jnp.zeros_like(acc_ref)

    acc_ref[...] += jnp.sum(h, axis=0, keepdims=True)

    @pl.when(i == pl.num_programs(0) - 1)
    def _():
        pool = acc_ref[...] * jnp.float32(1.0 / N)
        pool_ref[...] = pool
        z = jnp.dot(pool, wfc_ref[...], preferred_element_type=jnp.float32)
        out_ref[...] = jax.nn.sigmoid(z + bfc_ref[...])


def _fin(p, bias, wfc, bfc):
    RB = 2000
    return pl.pallas_call(
        _fin_body,
        grid=(N // RB,),
        in_specs=[pl.BlockSpec((2, RB, AUGW), lambda i: (0, i, 0)),
                  pl.BlockSpec((1, C), lambda i: (0, 0)),
                  pl.BlockSpec((C, 2), lambda i: (0, 0)),
                  pl.BlockSpec((1, 2), lambda i: (0, 0))],
        out_specs=[pl.BlockSpec((RB, C), lambda i: (i, 0)),
                   pl.BlockSpec((1, C), lambda i: (0, 0)),
                   pl.BlockSpec((1, 2), lambda i: (0, 0))],
        out_shape=[jax.ShapeDtypeStruct((N, C), jnp.float32),
                   jax.ShapeDtypeStruct((1, C), jnp.float32),
                   jax.ShapeDtypeStruct((1, 2), jnp.float32)],
        scratch_shapes=[pltpu.VMEM((1, C), jnp.float32)],
    )(p, bias, wfc, bfc)


def kernel(x, theta0, bias0, theta1, bias1, W_fc, b_fc):
    xpad = jnp.pad(x, ((0, NP - N), (0, 0)))
    xpadT = xpad.T
    parts = []
    for lo, nr, rows_valid, rpw in ((0, 2560, 2560, 80),
                                    (2560, 2560, 2560, 80),
                                    (5120, 2560, 2560, 80),
                                    (7680, 2560, 2320, 73)):
        dq, auxq = _build_dist(xpad[lo:lo + nr], xpadT, nr)
        parts.append(_select(dq, auxq, rows_valid, rpw))
    nn = jnp.concatenate(parts, axis=0)
    zz = jnp.zeros((N, AUGW), jnp.float32)
    x1 = _mm(x, theta0)
    p1 = _layer(x1, nn, zz)
    x2 = _mid(p1, bias0.reshape(1, C), theta1)
    p2 = _layer(x2, nn, zz)
    feats, pool, out2 = _fin(p2, bias1.reshape(1, C), W_fc,
                             b_fc.reshape(1, 2))
    return (out2[0], feats, pool)
